# baseline (speedup 1.0000x reference)
import jax
import jax.numpy as jnp
from jax.experimental import pallas as pl
from jax.experimental.pallas import tpu as pltpu

N, C = 16384, 2048
CPG = 16
T = 5
EPS = 1e-5
SW = 256
S = C // SW
BN = 512
NB = N // BN


def _fused_kernel(scal_ref, x_ref, w2_ref, b2_ref, o_ref,
                  sxx_scr, sx_scr, wfin_scr, beff_scr):
    i = pl.program_id(0)

    @pl.when(i == 0)
    def _():
        sxx_scr[...] = jnp.zeros_like(sxx_scr)
        sx_scr[...] = jnp.zeros_like(sx_scr)

    @pl.when(i < NB)
    def _():
        for s in range(S):
            xs = x_ref[:, s * SW:(s + 1) * SW]
            sxx_scr[s] += jax.lax.dot_general(
                xs, xs, (((0,), (0,)), ((), ())),
                preferred_element_type=jnp.float32)
            sx_scr[s] += jnp.sum(xs, axis=0, keepdims=True)

    @pl.when(i == NB - 1)
    def _():
        a = scal_ref[0]
        b = scal_ref[1]
        vw0 = scal_ref[2]
        ii = jax.lax.broadcasted_iota(jnp.int32, (SW, SW), 0)
        jj = jax.lax.broadcasted_iota(jnp.int32, (SW, SW), 1)
        eye = (ii == jj).astype(jnp.float32)
        mask = ((ii // CPG) == (jj // CPG)).astype(jnp.float32)
        inv_n = jnp.float32(1.0 / N)

        for s in range(S):
            mu = sx_scr[s] * inv_n
            exx = sxx_scr[s] * inv_n
            outer = jax.lax.dot_general(
                mu, mu, (((0,), (0,)), ((), ())),
                preferred_element_type=jnp.float32)
            cov = vw0 * (mask * (exx - outer)) + EPS * eye
            tr_elem = jnp.sum(cov * eye, axis=0, keepdims=True)
            group_tr = jnp.dot(tr_elem, mask,
                               preferred_element_type=jnp.float32)
            r = 1.0 / group_tr
            covNm = cov * (-0.5 * r)
            P = 1.5 * eye + covNm
            for _ in range(T - 1):
                P2 = jnp.dot(P, P, preferred_element_type=jnp.float32)
                P3 = jnp.dot(P2, P, preferred_element_type=jnp.float32)
                P = 1.5 * P + jnp.dot(P3, covNm,
                                      preferred_element_type=jnp.float32)
            wm = P * jnp.sqrt(r)
            b0 = wm * w2_ref[s]
            wfin_scr[s] = a * b0
            beff_scr[s] = b2_ref[s] - b * jnp.dot(
                mu, b0, preferred_element_type=jnp.float32)

    @pl.when(i >= NB)
    def _():
        for s in range(S):
            xs = x_ref[:, s * SW:(s + 1) * SW]
            o_ref[:, s * SW:(s + 1) * SW] = jnp.dot(
                xs, wfin_scr[s], preferred_element_type=jnp.float32
            ) + beff_scr[s]


def kernel(x, sw_mean_weight, sw_var_weight, weight, bias):
    mw = jax.nn.softmax(sw_mean_weight)
    vw = jax.nn.softmax(sw_var_weight)
    scal = jnp.stack([1.0 - mw[1], mw[0], vw[0]]).astype(jnp.float32)
    w2 = weight.reshape(S, 1, SW)
    b2 = bias.reshape(S, 1, SW)

    y = pl.pallas_call(
        _fused_kernel,
        grid=(2 * NB,),
        in_specs=[
            pl.BlockSpec(memory_space=pltpu.SMEM),
            pl.BlockSpec((BN, C), lambda i: (jax.lax.rem(i, NB), 0)),
            pl.BlockSpec((S, 1, SW), lambda i: (0, 0, 0)),
            pl.BlockSpec((S, 1, SW), lambda i: (0, 0, 0)),
        ],
        out_specs=pl.BlockSpec(
            (BN, C), lambda i: (jnp.where(i < NB, 0, i - NB), 0)),
        out_shape=jax.ShapeDtypeStruct((N, C), jnp.float32),
        scratch_shapes=[
            pltpu.VMEM((S, SW, SW), jnp.float32),
            pltpu.VMEM((S, 1, SW), jnp.float32),
            pltpu.VMEM((S, SW, SW), jnp.float32),
            pltpu.VMEM((S, 1, SW), jnp.float32),
        ],
        compiler_params=pltpu.CompilerParams(
            dimension_semantics=("arbitrary",),
            vmem_limit_bytes=48 * 1024 * 1024,
        ),
        name="sw_fused",
    )(scal, x, w2, b2)
    return y


# speedup vs baseline: 1.0604x; 1.0604x over previous
import jax
import jax.numpy as jnp
from jax.experimental import pallas as pl
from jax.experimental.pallas import tpu as pltpu

N, C = 16384, 2048
CPG = 16
T = 5
EPS = 1e-5
SW = 256
S = C // SW
BN = 1024
NB = N // BN


def _fused_kernel(scal_ref, x_ref, w2_ref, b2_ref, o_ref,
                  sxx_scr, sx_scr, wfin_scr, beff_scr):
    i = pl.program_id(0)

    @pl.when(i == 0)
    def _():
        sxx_scr[...] = jnp.zeros_like(sxx_scr)
        sx_scr[...] = jnp.zeros_like(sx_scr)

    @pl.when(i < NB)
    def _():
        for s in range(S):
            xs = x_ref[:, s * SW:(s + 1) * SW]
            sxx_scr[s] += jax.lax.dot_general(
                xs, xs, (((0,), (0,)), ((), ())),
                preferred_element_type=jnp.float32)
            sx_scr[s] += jnp.sum(xs, axis=0, keepdims=True)

    @pl.when(i == NB - 1)
    def _():
        a = scal_ref[0]
        b = scal_ref[1]
        vw0 = scal_ref[2]
        ii = jax.lax.broadcasted_iota(jnp.int32, (SW, SW), 0)
        jj = jax.lax.broadcasted_iota(jnp.int32, (SW, SW), 1)
        eye = (ii == jj).astype(jnp.float32)
        mask = ((ii // CPG) == (jj // CPG)).astype(jnp.float32)
        inv_n = jnp.float32(1.0 / N)

        for s in range(S):
            mu = sx_scr[s] * inv_n
            exx = sxx_scr[s] * inv_n
            outer = jax.lax.dot_general(
                mu, mu, (((0,), (0,)), ((), ())),
                preferred_element_type=jnp.float32)
            cov = vw0 * (mask * (exx - outer)) + EPS * eye
            tr_elem = jnp.sum(cov * eye, axis=0, keepdims=True)
            group_tr = jnp.dot(tr_elem, mask,
                               preferred_element_type=jnp.float32)
            r = 1.0 / group_tr
            covNm = cov * (-0.5 * r)
            covNb = covNm.astype(jnp.bfloat16)
            P = 1.5 * eye + covNm
            for _ in range(T - 1):
                Pb = P.astype(jnp.bfloat16)
                P2 = jnp.dot(Pb, Pb, preferred_element_type=jnp.float32)
                P3 = jnp.dot(P2.astype(jnp.bfloat16), Pb,
                             preferred_element_type=jnp.float32)
                P = 1.5 * P + jnp.dot(P3.astype(jnp.bfloat16), covNb,
                                      preferred_element_type=jnp.float32)
            wm = P * jnp.sqrt(r)
            b0 = wm * w2_ref[s]
            wfin_scr[s] = a * b0
            beff_scr[s] = b2_ref[s] - b * jnp.dot(
                mu, b0, preferred_element_type=jnp.float32)

    @pl.when(i >= NB)
    def _():
        for s in range(S):
            xs = x_ref[:, s * SW:(s + 1) * SW]
            o_ref[:, s * SW:(s + 1) * SW] = jnp.dot(
                xs, wfin_scr[s], preferred_element_type=jnp.float32
            ) + beff_scr[s]


def kernel(x, sw_mean_weight, sw_var_weight, weight, bias):
    mw = jax.nn.softmax(sw_mean_weight)
    vw = jax.nn.softmax(sw_var_weight)
    scal = jnp.stack([1.0 - mw[1], mw[0], vw[0]]).astype(jnp.float32)
    w2 = weight.reshape(S, 1, SW)
    b2 = bias.reshape(S, 1, SW)

    y = pl.pallas_call(
        _fused_kernel,
        grid=(2 * NB,),
        in_specs=[
            pl.BlockSpec(memory_space=pltpu.SMEM),
            pl.BlockSpec((BN, C), lambda i: (jax.lax.rem(i, NB), 0)),
            pl.BlockSpec((S, 1, SW), lambda i: (0, 0, 0)),
            pl.BlockSpec((S, 1, SW), lambda i: (0, 0, 0)),
        ],
        out_specs=pl.BlockSpec(
            (BN, C), lambda i: (jnp.where(i < NB, 0, i - NB), 0)),
        out_shape=jax.ShapeDtypeStruct((N, C), jnp.float32),
        scratch_shapes=[
            pltpu.VMEM((S, SW, SW), jnp.float32),
            pltpu.VMEM((S, 1, SW), jnp.float32),
            pltpu.VMEM((S, SW, SW), jnp.float32),
            pltpu.VMEM((S, 1, SW), jnp.float32),
        ],
        compiler_params=pltpu.CompilerParams(
            dimension_semantics=("arbitrary",),
            vmem_limit_bytes=48 * 1024 * 1024,
        ),
        name="sw_fused",
    )(scal, x, w2, b2)
    return y


# speedup vs baseline: 1.0627x; 1.0022x over previous
import jax
import jax.numpy as jnp
from jax.experimental import pallas as pl
from jax.experimental.pallas import tpu as pltpu

N, C = 16384, 2048
CPG = 16
T = 5
EPS = 1e-5
SW = 256
S = C // SW
BN = 1024
NB = N // BN


def _fused_kernel(scal_ref, x_ref, w2_ref, b2_ref, o_ref,
                  sxx_scr, sx_scr, wfin_scr, beff_scr):
    i = pl.program_id(0)

    @pl.when(i == 0)
    def _():
        sxx_scr[...] = jnp.zeros_like(sxx_scr)
        sx_scr[...] = jnp.zeros_like(sx_scr)

    @pl.when(i < NB)
    def _():
        for s in range(S):
            xs = x_ref[:, s * SW:(s + 1) * SW]
            sxx_scr[s] += jax.lax.dot_general(
                xs, xs, (((0,), (0,)), ((), ())),
                preferred_element_type=jnp.float32)
            sx_scr[s] += jnp.sum(xs, axis=0, keepdims=True)

    @pl.when(i == NB - 1)
    def _():
        a = scal_ref[0]
        b = scal_ref[1]
        vw0 = scal_ref[2]
        ii = jax.lax.broadcasted_iota(jnp.int32, (SW, SW), 0)
        jj = jax.lax.broadcasted_iota(jnp.int32, (SW, SW), 1)
        eye = (ii == jj).astype(jnp.float32)
        mask = ((ii // CPG) == (jj // CPG)).astype(jnp.float32)
        inv_n = jnp.float32(1.0 / N)

        for s in range(S):
            mu = sx_scr[s] * inv_n
            exx = sxx_scr[s] * inv_n
            outer = jax.lax.dot_general(
                mu, mu, (((0,), (0,)), ((), ())),
                preferred_element_type=jnp.float32)
            cov = vw0 * (mask * (exx - outer)) + EPS * eye
            tr_elem = jnp.sum(cov * eye, axis=0, keepdims=True)
            group_tr = jnp.dot(tr_elem, mask,
                               preferred_element_type=jnp.float32)
            r = 1.0 / group_tr
            covNm = cov * (-0.5 * r)
            P = 1.5 * eye + covNm
            for _ in range(T - 1):
                P2 = jnp.dot(P, P, preferred_element_type=jnp.float32)
                P3 = jnp.dot(P2, P, preferred_element_type=jnp.float32)
                P = 1.5 * P + jnp.dot(P3, covNm,
                                      preferred_element_type=jnp.float32)
            wm = P * jnp.sqrt(r)
            b0 = wm * w2_ref[s]
            wfin_scr[s] = a * b0
            beff_scr[s] = b2_ref[s] - b * jnp.dot(
                mu, b0, preferred_element_type=jnp.float32)

    @pl.when(i >= NB)
    def _():
        for s in range(S):
            xs = x_ref[:, s * SW:(s + 1) * SW]
            o_ref[:, s * SW:(s + 1) * SW] = jnp.dot(
                xs, wfin_scr[s], preferred_element_type=jnp.float32
            ) + beff_scr[s]


def kernel(x, sw_mean_weight, sw_var_weight, weight, bias):
    mw = jax.nn.softmax(sw_mean_weight)
    vw = jax.nn.softmax(sw_var_weight)
    scal = jnp.stack([1.0 - mw[1], mw[0], vw[0]]).astype(jnp.float32)
    w2 = weight.reshape(S, 1, SW)
    b2 = bias.reshape(S, 1, SW)

    y = pl.pallas_call(
        _fused_kernel,
        grid=(2 * NB,),
        in_specs=[
            pl.BlockSpec(memory_space=pltpu.SMEM),
            pl.BlockSpec((BN, C), lambda i: (jax.lax.rem(i, NB), 0)),
            pl.BlockSpec((S, 1, SW), lambda i: (0, 0, 0)),
            pl.BlockSpec((S, 1, SW), lambda i: (0, 0, 0)),
        ],
        out_specs=pl.BlockSpec(
            (BN, C), lambda i: (jnp.where(i < NB, 0, i - NB), 0)),
        out_shape=jax.ShapeDtypeStruct((N, C), jnp.float32),
        scratch_shapes=[
            pltpu.VMEM((S, SW, SW), jnp.float32),
            pltpu.VMEM((S, 1, SW), jnp.float32),
            pltpu.VMEM((S, SW, SW), jnp.float32),
            pltpu.VMEM((S, 1, SW), jnp.float32),
        ],
        compiler_params=pltpu.CompilerParams(
            dimension_semantics=("arbitrary",),
            vmem_limit_bytes=48 * 1024 * 1024,
        ),
        name="sw_fused",
    )(scal, x, w2, b2)
    return y


# speedup vs baseline: 1.0903x; 1.0260x over previous
import jax
import jax.numpy as jnp
from jax.experimental import pallas as pl
from jax.experimental.pallas import tpu as pltpu

N, C = 16384, 2048
CPG = 16
T = 5
EPS = 1e-5
SW = 256
S = C // SW
BN = 1024
NB = N // BN


def _fused_kernel(scal_ref, x_ref, w2_ref, b2_ref, o_ref,
                  sxx_scr, sx_scr, wfin_scr, beff_scr):
    i = pl.program_id(0)

    @pl.when(i == 0)
    def _():
        sxx_scr[...] = jnp.zeros_like(sxx_scr)
        sx_scr[...] = jnp.zeros_like(sx_scr)

    @pl.when(i < NB)
    def _():
        for s in range(S):
            xs = x_ref[:, s * SW:(s + 1) * SW]
            xb = xs.astype(jnp.bfloat16)
            sxx_scr[s] += jax.lax.dot_general(
                xb, xb, (((0,), (0,)), ((), ())),
                preferred_element_type=jnp.float32)
            sx_scr[s] += jnp.sum(xs, axis=0, keepdims=True)

    @pl.when(i == NB - 1)
    def _():
        a = scal_ref[0]
        b = scal_ref[1]
        vw0 = scal_ref[2]
        ii = jax.lax.broadcasted_iota(jnp.int32, (SW, SW), 0)
        jj = jax.lax.broadcasted_iota(jnp.int32, (SW, SW), 1)
        eye = (ii == jj).astype(jnp.float32)
        mask = ((ii // CPG) == (jj // CPG)).astype(jnp.float32)
        inv_n = jnp.float32(1.0 / N)

        for s in range(S):
            mu = sx_scr[s] * inv_n
            exx = sxx_scr[s] * inv_n
            outer = jax.lax.dot_general(
                mu, mu, (((0,), (0,)), ((), ())),
                preferred_element_type=jnp.float32)
            cov = vw0 * (mask * (exx - outer)) + EPS * eye
            tr_elem = jnp.sum(cov * eye, axis=0, keepdims=True)
            group_tr = jnp.dot(tr_elem, mask,
                               preferred_element_type=jnp.float32)
            r = 1.0 / group_tr
            covNm = cov * (-0.5 * r)
            P = 1.5 * eye + covNm
            for _ in range(T - 1):
                P2 = jnp.dot(P, P, preferred_element_type=jnp.float32)
                P3 = jnp.dot(P2, P, preferred_element_type=jnp.float32)
                P = 1.5 * P + jnp.dot(P3, covNm,
                                      preferred_element_type=jnp.float32)
            wm = P * jnp.sqrt(r)
            b0 = wm * w2_ref[s]
            wfin_scr[s] = a * b0
            beff_scr[s] = b2_ref[s] - b * jnp.dot(
                mu, b0, preferred_element_type=jnp.float32)

    @pl.when(i >= NB)
    def _():
        for s in range(S):
            xs = x_ref[:, s * SW:(s + 1) * SW]
            o_ref[:, s * SW:(s + 1) * SW] = jnp.dot(
                xs, wfin_scr[s], preferred_element_type=jnp.float32
            ) + beff_scr[s]


def kernel(x, sw_mean_weight, sw_var_weight, weight, bias):
    mw = jax.nn.softmax(sw_mean_weight)
    vw = jax.nn.softmax(sw_var_weight)
    scal = jnp.stack([1.0 - mw[1], mw[0], vw[0]]).astype(jnp.float32)
    w2 = weight.reshape(S, 1, SW)
    b2 = bias.reshape(S, 1, SW)

    y = pl.pallas_call(
        _fused_kernel,
        grid=(2 * NB,),
        in_specs=[
            pl.BlockSpec(memory_space=pltpu.SMEM),
            pl.BlockSpec((BN, C), lambda i: (jax.lax.rem(i, NB), 0)),
            pl.BlockSpec((S, 1, SW), lambda i: (0, 0, 0)),
            pl.BlockSpec((S, 1, SW), lambda i: (0, 0, 0)),
        ],
        out_specs=pl.BlockSpec(
            (BN, C), lambda i: (jnp.where(i < NB, 0, i - NB), 0)),
        out_shape=jax.ShapeDtypeStruct((N, C), jnp.float32),
        scratch_shapes=[
            pltpu.VMEM((S, SW, SW), jnp.float32),
            pltpu.VMEM((S, 1, SW), jnp.float32),
            pltpu.VMEM((S, SW, SW), jnp.float32),
            pltpu.VMEM((S, 1, SW), jnp.float32),
        ],
        compiler_params=pltpu.CompilerParams(
            dimension_semantics=("arbitrary",),
            vmem_limit_bytes=48 * 1024 * 1024,
        ),
        name="sw_fused",
    )(scal, x, w2, b2)
    return y


# speedup vs baseline: 1.1140x; 1.0217x over previous
import jax
import jax.numpy as jnp
from jax.experimental import pallas as pl
from jax.experimental.pallas import tpu as pltpu

N, C = 16384, 2048
CPG = 16
T = 5
EPS = 1e-5
SW = 256
S = C // SW
BN = 1024
NB = N // BN


def _fused_kernel(scal_ref, x_ref, w2_ref, b2_ref, o_ref,
                  sxx_scr, sx_scr, wfin_scr, beff_scr, xsave_scr):
    i = pl.program_id(0)

    @pl.when(i == 0)
    def _():
        sxx_scr[...] = jnp.zeros_like(sxx_scr)
        sx_scr[...] = jnp.zeros_like(sx_scr)

    @pl.when(i < NB)
    def _():
        for s in range(S):
            xs = x_ref[:, s * SW:(s + 1) * SW]
            xb = xs.astype(jnp.bfloat16)
            sxx_scr[s] += jax.lax.dot_general(
                xb, xb, (((0,), (0,)), ((), ())),
                preferred_element_type=jnp.float32)
            sx_scr[s] += jnp.sum(xs, axis=0, keepdims=True)

    @pl.when(i == NB - 3)
    def _():
        xsave_scr[0] = x_ref[...]

    @pl.when(i == NB - 2)
    def _():
        xsave_scr[1] = x_ref[...]

    @pl.when(i == NB - 1)
    def _():
        a = scal_ref[0]
        b = scal_ref[1]
        vw0 = scal_ref[2]
        ii = jax.lax.broadcasted_iota(jnp.int32, (SW, SW), 0)
        jj = jax.lax.broadcasted_iota(jnp.int32, (SW, SW), 1)
        eye = (ii == jj).astype(jnp.float32)
        mask = ((ii // CPG) == (jj // CPG)).astype(jnp.float32)
        inv_n = jnp.float32(1.0 / N)

        for s in range(S):
            mu = sx_scr[s] * inv_n
            exx = sxx_scr[s] * inv_n
            outer = jax.lax.dot_general(
                mu, mu, (((0,), (0,)), ((), ())),
                preferred_element_type=jnp.float32)
            cov = vw0 * (mask * (exx - outer)) + EPS * eye
            tr_elem = jnp.sum(cov * eye, axis=0, keepdims=True)
            group_tr = jnp.dot(tr_elem, mask,
                               preferred_element_type=jnp.float32)
            r = 1.0 / group_tr
            covNm = cov * (-0.5 * r)
            P = 1.5 * eye + covNm
            for _ in range(T - 1):
                P2 = jnp.dot(P, P, preferred_element_type=jnp.float32)
                P3 = jnp.dot(P2, P, preferred_element_type=jnp.float32)
                P = 1.5 * P + jnp.dot(P3, covNm,
                                      preferred_element_type=jnp.float32)
            wm = P * jnp.sqrt(r)
            b0 = wm * w2_ref[s]
            wfin_scr[s] = a * b0
            beff_scr[s] = b2_ref[s] - b * jnp.dot(
                mu, b0, preferred_element_type=jnp.float32)

    def _apply_from(src_ref):
        for s in range(S):
            xs = src_ref[:, s * SW:(s + 1) * SW]
            o_ref[:, s * SW:(s + 1) * SW] = jnp.dot(
                xs, wfin_scr[s], preferred_element_type=jnp.float32
            ) + beff_scr[s]

    @pl.when(jnp.logical_or(i == NB, i >= NB + 3))
    def _():
        _apply_from(x_ref)

    @pl.when(i == NB + 1)
    def _():
        _apply_from(xsave_scr.at[1])

    @pl.when(i == NB + 2)
    def _():
        _apply_from(xsave_scr.at[0])


def kernel(x, sw_mean_weight, sw_var_weight, weight, bias):
    mw = jax.nn.softmax(sw_mean_weight)
    vw = jax.nn.softmax(sw_var_weight)
    scal = jnp.stack([1.0 - mw[1], mw[0], vw[0]]).astype(jnp.float32)
    w2 = weight.reshape(S, 1, SW)
    b2 = bias.reshape(S, 1, SW)

    y = pl.pallas_call(
        _fused_kernel,
        grid=(2 * NB,),
        in_specs=[
            pl.BlockSpec(memory_space=pltpu.SMEM),
            pl.BlockSpec((BN, C), lambda i: (jnp.where(
                i < NB, i,
                jnp.where(i < NB + 3, NB - 1, i - NB - 3)), 0)),
            pl.BlockSpec((S, 1, SW), lambda i: (0, 0, 0)),
            pl.BlockSpec((S, 1, SW), lambda i: (0, 0, 0)),
        ],
        out_specs=pl.BlockSpec(
            (BN, C), lambda i: (jnp.where(
                i < NB, NB - 1,
                jnp.where(i < NB + 3, 2 * NB - 1 - i, i - NB - 3)), 0)),
        out_shape=jax.ShapeDtypeStruct((N, C), jnp.float32),
        scratch_shapes=[
            pltpu.VMEM((S, SW, SW), jnp.float32),
            pltpu.VMEM((S, 1, SW), jnp.float32),
            pltpu.VMEM((S, SW, SW), jnp.float32),
            pltpu.VMEM((S, 1, SW), jnp.float32),
            pltpu.VMEM((2, BN, C), jnp.float32),
        ],
        compiler_params=pltpu.CompilerParams(
            dimension_semantics=("arbitrary",),
            vmem_limit_bytes=56 * 1024 * 1024,
        ),
        name="sw_fused",
    )(scal, x, w2, b2)
    return y


# speedup vs baseline: 1.1548x; 1.0366x over previous
import jax
import jax.numpy as jnp
from jax.experimental import pallas as pl
from jax.experimental.pallas import tpu as pltpu

N, C = 16384, 2048
CPG = 16
T = 5
EPS = 1e-5
SW = 256
S = C // SW
BN = 1024
NB = N // BN


def _fused_kernel(scal_ref, x_ref, w2_ref, b2_ref, o_ref,
                  sxx_scr, sx_scr, wfin_scr, wfinb_scr, beff_scr,
                  xsave_scr):
    i = pl.program_id(0)

    @pl.when(i == 0)
    def _():
        sxx_scr[...] = jnp.zeros_like(sxx_scr)
        sx_scr[...] = jnp.zeros_like(sx_scr)

    @pl.when(i < NB)
    def _():
        for s in range(S):
            xs = x_ref[:, s * SW:(s + 1) * SW]
            xb = xs.astype(jnp.bfloat16)
            sxx_scr[s] += jax.lax.dot_general(
                xb, xb, (((0,), (0,)), ((), ())),
                preferred_element_type=jnp.float32)
            sx_scr[s] += jnp.sum(xs, axis=0, keepdims=True)

    for k in range(4):
        @pl.when(i == NB - 5 + k)
        def _(k=k):
            xsave_scr[k] = x_ref[...].astype(jnp.bfloat16)

    @pl.when(i == NB - 1)
    def _():
        a = scal_ref[0]
        b = scal_ref[1]
        vw0 = scal_ref[2]
        ii = jax.lax.broadcasted_iota(jnp.int32, (SW, SW), 0)
        jj = jax.lax.broadcasted_iota(jnp.int32, (SW, SW), 1)
        eye = (ii == jj).astype(jnp.float32)
        mask = ((ii // CPG) == (jj // CPG)).astype(jnp.float32)
        inv_n = jnp.float32(1.0 / N)

        for s in range(S):
            mu = sx_scr[s] * inv_n
            exx = sxx_scr[s] * inv_n
            outer = jax.lax.dot_general(
                mu, mu, (((0,), (0,)), ((), ())),
                preferred_element_type=jnp.float32)
            cov = vw0 * (mask * (exx - outer)) + EPS * eye
            tr_elem = jnp.sum(cov * eye, axis=0, keepdims=True)
            group_tr = jnp.dot(tr_elem, mask,
                               preferred_element_type=jnp.float32)
            r = 1.0 / group_tr
            covNm = cov * (-0.5 * r)
            P = 1.5 * eye + covNm
            for _ in range(T - 1):
                P2 = jnp.dot(P, P, preferred_element_type=jnp.float32)
                P3 = jnp.dot(P2, P, preferred_element_type=jnp.float32)
                P = 1.5 * P + jnp.dot(P3, covNm,
                                      preferred_element_type=jnp.float32)
            wm = P * jnp.sqrt(r)
            b0 = wm * w2_ref[s]
            wfin_scr[s] = a * b0
            wfinb_scr[s] = (a * b0).astype(jnp.bfloat16)
            beff_scr[s] = b2_ref[s] - b * jnp.dot(
                mu, b0, preferred_element_type=jnp.float32)

    def _apply_from(src_ref):
        for s in range(S):
            xs = src_ref[:, s * SW:(s + 1) * SW]
            o_ref[:, s * SW:(s + 1) * SW] = jnp.dot(
                xs, wfin_scr[s], preferred_element_type=jnp.float32
            ) + beff_scr[s]

    def _apply_bf16(src_ref):
        for s in range(S):
            xs = src_ref[:, s * SW:(s + 1) * SW]
            o_ref[:, s * SW:(s + 1) * SW] = jnp.dot(
                xs, wfinb_scr[s], preferred_element_type=jnp.float32
            ) + beff_scr[s]

    @pl.when(jnp.logical_or(i == NB, i >= NB + 5))
    def _():
        _apply_from(x_ref)

    for k in range(4):
        @pl.when(i == NB + 1 + k)
        def _(k=k):
            _apply_bf16(xsave_scr.at[3 - k])


def kernel(x, sw_mean_weight, sw_var_weight, weight, bias):
    mw = jax.nn.softmax(sw_mean_weight)
    vw = jax.nn.softmax(sw_var_weight)
    scal = jnp.stack([1.0 - mw[1], mw[0], vw[0]]).astype(jnp.float32)
    w2 = weight.reshape(S, 1, SW)
    b2 = bias.reshape(S, 1, SW)

    y = pl.pallas_call(
        _fused_kernel,
        grid=(2 * NB,),
        in_specs=[
            pl.BlockSpec(memory_space=pltpu.SMEM),
            pl.BlockSpec((BN, C), lambda i: (jnp.where(
                i < NB, i,
                jnp.where(i < NB + 5, NB - 1, i - NB - 5)), 0)),
            pl.BlockSpec((S, 1, SW), lambda i: (0, 0, 0)),
            pl.BlockSpec((S, 1, SW), lambda i: (0, 0, 0)),
        ],
        out_specs=pl.BlockSpec(
            (BN, C), lambda i: (jnp.where(
                i < NB, NB - 1,
                jnp.where(i < NB + 5, 2 * NB - 1 - i, i - NB - 5)), 0)),
        out_shape=jax.ShapeDtypeStruct((N, C), jnp.float32),
        scratch_shapes=[
            pltpu.VMEM((S, SW, SW), jnp.float32),
            pltpu.VMEM((S, 1, SW), jnp.float32),
            pltpu.VMEM((S, SW, SW), jnp.float32),
            pltpu.VMEM((S, SW, SW), jnp.bfloat16),
            pltpu.VMEM((S, 1, SW), jnp.float32),
            pltpu.VMEM((4, BN, C), jnp.bfloat16),
        ],
        compiler_params=pltpu.CompilerParams(
            dimension_semantics=("arbitrary",),
            vmem_limit_bytes=56 * 1024 * 1024,
        ),
        name="sw_fused",
    )(scal, x, w2, b2)
    return y
